# Initial kernel scaffold; baseline (speedup 1.0000x reference)
import jax
import jax.numpy as jnp
from jax.experimental import pallas as pl


def kernel(coordinates, W_tlx, W_brx, W_tly, W_bry, W_w, W_h):
    raise NotImplementedError("write your pallas kernel here")


# speedup vs baseline: 2.0165x; 2.0165x over previous
import functools

import jax
import jax.numpy as jnp
from jax import lax
from jax.experimental import pallas as pl
from jax.experimental.pallas import tpu as pltpu
from jax.experimental.pallas import tpu_sc as plsc

B, L = 4, 8192
BL = B * L
V = 1024
D = 1024
HD = D // 2

NC, NS = 2, 16
NW = NC * NS
TPW = BL // NW
T = 32
NCHUNK = TPW // T
R = 2 * T
IDXLEN = 2 * TPW


def _accum(acc, stg, r0):
    def row_body(r, carry):
        for j in range(HD // 16):
            plsc.addupdate(acc.at[r0 + r, pl.ds(16 * j, 16)],
                           stg[r, pl.ds(16 * j, 16)])
        return carry
    lax.fori_loop(0, R, row_body, 0)


def _body(tab, x0h, y0h, x1h, y1h, out, xs, ys, wh, idx, acc, s0, s1,
          sem0, sem1, sem2, sem3):
    wid = lax.axis_index("s") * NC + lax.axis_index("c")
    tok0 = wid * TPW
    lane = lax.iota(jnp.int32, 16)
    half = lane & 1
    tokl = lane >> 1
    off_half = half * 4096

    pltpu.sync_copy(x0h.at[pl.ds(tok0, TPW)], xs.at[pl.ds(0, TPW)])
    pltpu.sync_copy(x1h.at[pl.ds(tok0, TPW)], xs.at[pl.ds(TPW, TPW)])
    pltpu.sync_copy(y0h.at[pl.ds(tok0, TPW)], ys.at[pl.ds(0, TPW)])
    pltpu.sync_copy(y1h.at[pl.ds(tok0, TPW)], ys.at[pl.ds(TPW, TPW)])

    def wh_body(g, carry):
        wh[pl.ds(16 * g, 16)] = (
            xs[pl.ds(TPW + 16 * g, 16)] - xs[pl.ds(16 * g, 16)])
        wh[pl.ds(TPW + 16 * g, 16)] = (
            ys[pl.ds(TPW + 16 * g, 16)] - ys[pl.ds(16 * g, 16)])
        return carry
    lax.fori_loop(0, TPW // 16, wh_body, 0)

    def idx_body(g, carry):
        base = jnp.full((16,), 8 * g, jnp.int32)
        src_pair = base + tokl + half * TPW
        src_same = base + tokl
        idx[0, pl.ds(16 * g, 16)] = plsc.load_gather(xs, [src_pair]) + off_half
        idx[1, pl.ds(16 * g, 16)] = (
            plsc.load_gather(ys, [src_pair]) + off_half + 1024)
        idx[2, pl.ds(16 * g, 16)] = (
            plsc.load_gather(wh, [src_same]) + off_half + 2048)
        idx[3, pl.ds(16 * g, 16)] = (
            plsc.load_gather(wh, [src_same + TPW]) + off_half + 3072)
        return carry
    lax.fori_loop(0, IDXLEN // 16, idx_body, 0)

    def chunk_body(ci, carry):
        e0 = 2 * ci * T
        d0 = pltpu.async_copy(tab.at[idx.at[0, pl.ds(e0, R)]], acc, sem0)
        d1 = pltpu.async_copy(tab.at[idx.at[1, pl.ds(e0, R)]], s0, sem1)
        d2 = pltpu.async_copy(tab.at[idx.at[2, pl.ds(e0, R)]], s1, sem2)
        d0.wait()
        d1.wait()
        _accum(acc, s0, 0)
        d3 = pltpu.async_copy(tab.at[idx.at[3, pl.ds(e0, R)]], s0, sem3)
        d2.wait()
        _accum(acc, s1, 0)
        d3.wait()
        _accum(acc, s0, 0)
        pltpu.sync_copy(acc, out.at[pl.ds(2 * tok0 + e0, R)])
        return carry

    lax.fori_loop(0, NCHUNK, chunk_body, 0)


@jax.jit
def _spatial_sc(tab, x0, y0, x1, y1):
    mesh = plsc.VectorSubcoreMesh(
        core_axis_name="c", subcore_axis_name="s", num_cores=NC, num_subcores=NS
    )
    return pl.kernel(
        _body,
        out_type=jax.ShapeDtypeStruct((2 * BL, HD), jnp.float32),
        mesh=mesh,
        scratch_types=[
            pltpu.VMEM((2 * TPW,), jnp.int32),
            pltpu.VMEM((2 * TPW,), jnp.int32),
            pltpu.VMEM((2 * TPW,), jnp.int32),
            pltpu.VMEM((4, IDXLEN), jnp.int32),
            pltpu.VMEM((R, HD), jnp.float32),
            pltpu.VMEM((R, HD), jnp.float32),
            pltpu.VMEM((R, HD), jnp.float32),
            pltpu.SemaphoreType.DMA,
            pltpu.SemaphoreType.DMA,
            pltpu.SemaphoreType.DMA,
            pltpu.SemaphoreType.DMA,
        ],
        compiler_params=pltpu.CompilerParams(needs_layout_passes=False),
    )(tab, x0, y0, x1, y1)


def kernel(coordinates, W_tlx, W_brx, W_tly, W_bry, W_w, W_h):
    tab = jnp.concatenate(
        [W_tlx, W_tly, W_w[:, :HD], W_h[:, :HD],
         W_brx, W_bry, W_w[:, HD:], W_h[:, HD:]], axis=0
    )
    c = coordinates.reshape(BL, 4)
    x0 = c[:, 0]
    y0 = c[:, 1]
    x1 = c[:, 2]
    y1 = c[:, 3]
    out = _spatial_sc(tab, x0, y0, x1, y1)
    return out.reshape(B, L, D)


# speedup vs baseline: 2.4441x; 1.2121x over previous
import jax
import jax.numpy as jnp
from jax import lax
from jax.experimental import pallas as pl
from jax.experimental.pallas import tpu as pltpu
from jax.experimental.pallas import tpu_sc as plsc

B, L = 4, 8192
BL = B * L
V = 1024
D = 1024
HD = D // 2

NC, NS = 2, 16
NW = NC * NS
TPW = BL // NW
T = 32
NCHUNK = TPW // T


def _accum(acc, stg):
    def row_body(r, carry):
        for j in range(D // 16):
            plsc.addupdate(acc.at[r, pl.ds(16 * j, 16)],
                           stg[r, pl.ds(16 * j, 16)])
        return carry
    lax.fori_loop(0, T, row_body, 0)


def _body(coords, tlx, brx, tly, bry, ww, wh_t, out,
          cb, ix, acc, s0, s1, sem0, sem1, sem2, sem3, sem4):
    wid = lax.axis_index("s") * NC + lax.axis_index("c")
    tok0 = wid * TPW
    lane = lax.iota(jnp.int32, 16)

    pltpu.sync_copy(coords.at[pl.ds(4 * tok0, 4 * TPW)], cb)

    def dec_body(g, carry):
        pos = 4 * (16 * g + lane)
        x0 = plsc.load_gather(cb, [pos])
        y0 = plsc.load_gather(cb, [pos + 1])
        x1 = plsc.load_gather(cb, [pos + 2])
        y1 = plsc.load_gather(cb, [pos + 3])
        sl = pl.ds(16 * g, 16)
        ix[0, sl] = x0
        ix[1, sl] = y0
        ix[2, sl] = x1
        ix[3, sl] = y1
        ix[4, sl] = x1 - x0
        ix[5, sl] = y1 - y0
        return carry
    lax.fori_loop(0, TPW // 16, dec_body, 0)

    def chunk_body(ci, carry):
        c0 = ci * T
        sl = pl.ds(c0, T)
        accL = acc.at[:, pl.ds(0, HD)]
        accR = acc.at[:, pl.ds(HD, HD)]
        s0L = s0.at[:, pl.ds(0, HD)]
        s0R = s0.at[:, pl.ds(HD, HD)]
        d1 = pltpu.async_copy(tlx.at[ix.at[0, sl]], accL, sem0)
        d2 = pltpu.async_copy(brx.at[ix.at[2, sl]], accR, sem1)
        d3 = pltpu.async_copy(tly.at[ix.at[1, sl]], s0L, sem2)
        d4 = pltpu.async_copy(bry.at[ix.at[3, sl]], s0R, sem3)
        d5 = pltpu.async_copy(ww.at[ix.at[4, sl]], s1, sem4)
        d1.wait()
        d2.wait()
        d3.wait()
        d4.wait()
        _accum(acc, s0)
        d6 = pltpu.async_copy(wh_t.at[ix.at[5, sl]], s0, sem2)
        d5.wait()
        _accum(acc, s1)
        d6.wait()
        _accum(acc, s0)
        pltpu.sync_copy(acc, out.at[pl.ds(tok0 + c0, T)])
        return carry

    lax.fori_loop(0, NCHUNK, chunk_body, 0)


@jax.jit
def _spatial_sc(coords, tlx, brx, tly, bry, ww, wh_t):
    mesh = plsc.VectorSubcoreMesh(
        core_axis_name="c", subcore_axis_name="s", num_cores=NC, num_subcores=NS
    )
    return pl.kernel(
        _body,
        out_type=jax.ShapeDtypeStruct((BL, D), jnp.float32),
        mesh=mesh,
        scratch_types=[
            pltpu.VMEM((4 * TPW,), jnp.int32),
            pltpu.VMEM((6, TPW), jnp.int32),
            pltpu.VMEM((T, D), jnp.float32),
            pltpu.VMEM((T, D), jnp.float32),
            pltpu.VMEM((T, D), jnp.float32),
            pltpu.SemaphoreType.DMA,
            pltpu.SemaphoreType.DMA,
            pltpu.SemaphoreType.DMA,
            pltpu.SemaphoreType.DMA,
            pltpu.SemaphoreType.DMA,
        ],
        compiler_params=pltpu.CompilerParams(needs_layout_passes=False),
    )(coords, tlx, brx, tly, bry, ww, wh_t)


def kernel(coordinates, W_tlx, W_brx, W_tly, W_bry, W_w, W_h):
    out = _spatial_sc(coordinates.reshape(4 * BL), W_tlx, W_brx, W_tly,
                      W_bry, W_w, W_h)
    return out.reshape(B, L, D)


# speedup vs baseline: 2.5180x; 1.0302x over previous
import jax
import jax.numpy as jnp
from jax import lax
from jax.experimental import pallas as pl
from jax.experimental.pallas import tpu as pltpu
from jax.experimental.pallas import tpu_sc as plsc

B, L = 4, 8192
BL = B * L
V = 1024
D = 1024
HD = D // 2
HDP = HD // 2
DP = D // 2

NC, NS = 2, 16
NW = NC * NS
TPW = BL // NW
T = 16
NCHUNK = TPW // T
NPAIR = NCHUNK // 2


def _issue_gathers(tabs, ix, sl, stg, sems):
    tlx, brx, tly, bry, ww, wh_t = tabs
    c1, c2, sw, sh = stg
    pltpu.async_copy(tlx.at[ix.at[0, sl]], c1.at[:, pl.ds(0, HDP)], sems[0])
    pltpu.async_copy(brx.at[ix.at[2, sl]], c1.at[:, pl.ds(HDP, HDP)], sems[1])
    pltpu.async_copy(tly.at[ix.at[1, sl]], c2.at[:, pl.ds(0, HDP)], sems[2])
    pltpu.async_copy(bry.at[ix.at[3, sl]], c2.at[:, pl.ds(HDP, HDP)], sems[3])
    pltpu.async_copy(ww.at[ix.at[4, sl]], sw, sems[4])
    pltpu.async_copy(wh_t.at[ix.at[5, sl]], sh, sems[5])


def _wait_gathers(tabs, ix, stg, sems):
    tlx, brx, tly, bry, ww, wh_t = tabs
    c1, c2, sw, sh = stg
    sl = pl.ds(0, T)
    pltpu.make_async_copy(
        tlx.at[ix.at[0, sl]], c1.at[:, pl.ds(0, HDP)], sems[0]).wait()
    pltpu.make_async_copy(
        brx.at[ix.at[2, sl]], c1.at[:, pl.ds(HDP, HDP)], sems[1]).wait()
    pltpu.make_async_copy(
        tly.at[ix.at[1, sl]], c2.at[:, pl.ds(0, HDP)], sems[2]).wait()
    pltpu.make_async_copy(
        bry.at[ix.at[3, sl]], c2.at[:, pl.ds(HDP, HDP)], sems[3]).wait()
    pltpu.make_async_copy(ww.at[ix.at[4, sl]], sw, sems[4]).wait()
    pltpu.make_async_copy(wh_t.at[ix.at[5, sl]], sh, sems[5]).wait()


def _accum_convert(stg, acc):
    c1, c2, sw, sh = stg

    def row_body(r, carry):
        rbase = r * D
        for j in range(DP // 16):
            slj = pl.ds(16 * j, 16)
            a = plsc.bitcast(c1[r, slj], jnp.bfloat16)
            b = plsc.bitcast(c2[r, slj], jnp.bfloat16)
            c = plsc.bitcast(sw[r, slj], jnp.bfloat16)
            d = plsc.bitcast(sh[r, slj], jnp.bfloat16)
            s = (a + b) + (c + d)
            lo, hi = plsc.unpack(s, format=plsc.PackFormat.INTERLEAVED)
            off0 = 16 * j if j < 16 else 256 + 16 * j
            acc[pl.ds(rbase + off0, 16)] = lo
            acc[pl.ds(rbase + off0 + 256, 16)] = hi
        return carry

    lax.fori_loop(0, T, row_body, 0)


def _body(coords, tlx, brx, tly, bry, ww, wh_t, out,
          cb, ix, accA, accB, c1A, c2A, swA, shA, c1B, c2B, swB, shB,
          *sems):
    tabs = (tlx, brx, tly, bry, ww, wh_t)
    stgA = (c1A, c2A, swA, shA)
    stgB = (c1B, c2B, swB, shB)
    semsA = sems[0:6]
    semsB = sems[6:12]
    semOA = sems[12]
    semOB = sems[13]

    wid = lax.axis_index("s") * NC + lax.axis_index("c")
    tok0 = wid * TPW
    lane = lax.iota(jnp.int32, 16)

    pltpu.sync_copy(coords.at[pl.ds(4 * tok0, 4 * TPW)], cb)

    def dec_body(g, carry):
        pos = 4 * (16 * g + lane)
        x0 = plsc.load_gather(cb, [pos])
        y0 = plsc.load_gather(cb, [pos + 1])
        x1 = plsc.load_gather(cb, [pos + 2])
        y1 = plsc.load_gather(cb, [pos + 3])
        sl = pl.ds(16 * g, 16)
        ix[0, sl] = x0
        ix[1, sl] = y0
        ix[2, sl] = x1
        ix[3, sl] = y1
        ix[4, sl] = x1 - x0
        ix[5, sl] = y1 - y0
        return carry
    lax.fori_loop(0, TPW // 16, dec_body, 0)

    _issue_gathers(tabs, ix, pl.ds(0, T), stgA, semsA)
    _issue_gathers(tabs, ix, pl.ds(T, T), stgB, semsB)

    def pair_body(k, carry):
        ca = 2 * k
        _wait_gathers(tabs, ix, stgA, semsA)

        @pl.when(k > 0)
        def _():
            pltpu.make_async_copy(
                accA, out.at[pl.ds(tok0 * D, T * D)], semOA).wait()

        _accum_convert(stgA, accA)
        pltpu.async_copy(
            accA, out.at[pl.ds((tok0 + ca * T) * D, T * D)], semOA)

        @pl.when(k < NPAIR - 1)
        def _():
            _issue_gathers(tabs, ix, pl.ds((ca + 2) * T, T), stgA, semsA)

        _wait_gathers(tabs, ix, stgB, semsB)

        @pl.when(k > 0)
        def _():
            pltpu.make_async_copy(
                accB, out.at[pl.ds(tok0 * D, T * D)], semOB).wait()

        _accum_convert(stgB, accB)
        pltpu.async_copy(
            accB, out.at[pl.ds((tok0 + (ca + 1) * T) * D, T * D)], semOB)

        @pl.when(k < NPAIR - 1)
        def _():
            _issue_gathers(tabs, ix, pl.ds((ca + 3) * T, T), stgB, semsB)

        return carry

    lax.fori_loop(0, NPAIR, pair_body, 0)

    pltpu.make_async_copy(accA, out.at[pl.ds(tok0 * D, T * D)], semOA).wait()
    pltpu.make_async_copy(accB, out.at[pl.ds(tok0 * D, T * D)], semOB).wait()


@jax.jit
def _spatial_sc(coords, tlx, brx, tly, bry, ww, wh_t):
    mesh = plsc.VectorSubcoreMesh(
        core_axis_name="c", subcore_axis_name="s", num_cores=NC, num_subcores=NS
    )
    return pl.kernel(
        _body,
        out_type=jax.ShapeDtypeStruct((BL * D,), jnp.float32),
        mesh=mesh,
        scratch_types=[
            pltpu.VMEM((4 * TPW,), jnp.int32),
            pltpu.VMEM((6, TPW), jnp.int32),
            pltpu.VMEM((T * D,), jnp.float32),
            pltpu.VMEM((T * D,), jnp.float32),
            pltpu.VMEM((T, HDP + HDP), jnp.int32),
            pltpu.VMEM((T, HDP + HDP), jnp.int32),
            pltpu.VMEM((T, DP), jnp.int32),
            pltpu.VMEM((T, DP), jnp.int32),
            pltpu.VMEM((T, HDP + HDP), jnp.int32),
            pltpu.VMEM((T, HDP + HDP), jnp.int32),
            pltpu.VMEM((T, DP), jnp.int32),
            pltpu.VMEM((T, DP), jnp.int32),
        ] + [pltpu.SemaphoreType.DMA] * 14,
        compiler_params=pltpu.CompilerParams(needs_layout_passes=False),
    )(coords, tlx, brx, tly, bry, ww, wh_t)


def _bf16_bits(w):
    u = lax.bitcast_convert_type(w, jnp.uint32)
    return (u + 0x7FFF + ((u >> 16) & 1)) >> 16


def _pack_half(h):
    r = _bf16_bits(h)
    return lax.bitcast_convert_type(
        r[:, :HDP] | (r[:, HDP:] << 16), jnp.int32)


def _pack_corner(w):
    return _pack_half(w)


def _pack_full(w):
    return jnp.concatenate([_pack_half(w[:, :HD]), _pack_half(w[:, HD:])],
                           axis=1)


def kernel(coordinates, W_tlx, W_brx, W_tly, W_bry, W_w, W_h):
    out = _spatial_sc(
        coordinates.reshape(4 * BL),
        _pack_corner(W_tlx), _pack_corner(W_brx), _pack_corner(W_tly),
        _pack_corner(W_bry), _pack_full(W_w), _pack_full(W_h))
    return out.reshape(B, L, D)


# speedup vs baseline: 2.6114x; 1.0371x over previous
import jax
import jax.numpy as jnp
from jax import lax
from jax.experimental import pallas as pl
from jax.experimental.pallas import tpu as pltpu
from jax.experimental.pallas import tpu_sc as plsc

B, L = 4, 8192
BL = B * L
V = 1024
D = 1024
HD = D // 2
HDP = HD // 2
DP = D // 2

NC, NS = 2, 16
NW = NC * NS
TPW = BL // NW
T = 16
NCHUNK = TPW // T
NPAIR = NCHUNK // 2


def _issue_gathers(tabs, ix, sl, stg, sems):
    tlx, brx, tly, bry, ww, wh_t = tabs
    c1, c2, sw, sh = stg
    pltpu.async_copy(tlx.at[ix.at[0, sl]], c1.at[:, pl.ds(0, HDP)], sems[0])
    pltpu.async_copy(brx.at[ix.at[2, sl]], c1.at[:, pl.ds(HDP, HDP)], sems[1])
    pltpu.async_copy(tly.at[ix.at[1, sl]], c2.at[:, pl.ds(0, HDP)], sems[2])
    pltpu.async_copy(bry.at[ix.at[3, sl]], c2.at[:, pl.ds(HDP, HDP)], sems[3])
    pltpu.async_copy(ww.at[ix.at[4, sl]], sw, sems[4])
    pltpu.async_copy(wh_t.at[ix.at[5, sl]], sh, sems[5])


def _wait_gathers(tabs, ix, stg, sems):
    tlx, brx, tly, bry, ww, wh_t = tabs
    c1, c2, sw, sh = stg
    sl = pl.ds(0, T)
    pltpu.make_async_copy(
        tlx.at[ix.at[0, sl]], c1.at[:, pl.ds(0, HDP)], sems[0]).wait()
    pltpu.make_async_copy(
        brx.at[ix.at[2, sl]], c1.at[:, pl.ds(HDP, HDP)], sems[1]).wait()
    pltpu.make_async_copy(
        tly.at[ix.at[1, sl]], c2.at[:, pl.ds(0, HDP)], sems[2]).wait()
    pltpu.make_async_copy(
        bry.at[ix.at[3, sl]], c2.at[:, pl.ds(HDP, HDP)], sems[3]).wait()
    pltpu.make_async_copy(ww.at[ix.at[4, sl]], sw, sems[4]).wait()
    pltpu.make_async_copy(wh_t.at[ix.at[5, sl]], sh, sems[5]).wait()


def _accum_convert(stg, acc):
    c1, c2, sw, sh = stg

    def row_body(r, carry):
        rbase = r * D
        for j in range(DP // 16):
            slj = pl.ds(16 * j, 16)
            a = plsc.bitcast(c1[r, slj], jnp.bfloat16)
            b = plsc.bitcast(c2[r, slj], jnp.bfloat16)
            c = plsc.bitcast(sw[r, slj], jnp.bfloat16)
            d = plsc.bitcast(sh[r, slj], jnp.bfloat16)
            s = (a + b) + (c + d)
            lo, hi = plsc.unpack(s, format=plsc.PackFormat.INTERLEAVED)
            off0 = 16 * j if j < 16 else 256 + 16 * j
            acc[pl.ds(rbase + off0, 16)] = lo
            acc[pl.ds(rbase + off0 + 256, 16)] = hi
        return carry

    lax.fori_loop(0, T, row_body, 0)


def _body(coords, tlx, brx, tly, bry, ww, wh_t, out,
          cb, ix, accA, accB, c1A, c2A, swA, shA, c1B, c2B, swB, shB,
          *sems):
    tabs = (tlx, brx, tly, bry, ww, wh_t)
    stgA = (c1A, c2A, swA, shA)
    stgB = (c1B, c2B, swB, shB)
    semsA = sems[0:6]
    semsB = sems[6:12]
    semOA = sems[12]
    semOB = sems[13]

    wid = lax.axis_index("s") * NC + lax.axis_index("c")
    tok0 = wid * TPW
    lane = lax.iota(jnp.int32, 16)

    pltpu.sync_copy(coords.at[pl.ds(4 * tok0, 4 * TPW)], cb)

    def dec_body(g, carry):
        pos = 4 * (16 * g + lane)
        x0 = plsc.load_gather(cb, [pos])
        y0 = plsc.load_gather(cb, [pos + 1])
        x1 = plsc.load_gather(cb, [pos + 2])
        y1 = plsc.load_gather(cb, [pos + 3])
        sl = pl.ds(16 * g, 16)
        ix[0, sl] = x0
        ix[1, sl] = y0
        ix[2, sl] = x1
        ix[3, sl] = y1
        ix[4, sl] = x1 - x0
        ix[5, sl] = y1 - y0
        return carry
    lax.fori_loop(0, TPW // 16, dec_body, 0)

    _issue_gathers(tabs, ix, pl.ds(0, T), stgA, semsA)
    _issue_gathers(tabs, ix, pl.ds(T, T), stgB, semsB)

    def pair_body(k, carry):
        ca = 2 * k
        _wait_gathers(tabs, ix, stgA, semsA)

        @pl.when(k > 0)
        def _():
            pltpu.make_async_copy(
                accA, out.at[pl.ds(tok0 * D, T * D)], semOA).wait()

        _accum_convert(stgA, accA)
        pltpu.async_copy(
            accA, out.at[pl.ds((tok0 + ca * T) * D, T * D)], semOA)

        @pl.when(k < NPAIR - 1)
        def _():
            _issue_gathers(tabs, ix, pl.ds((ca + 2) * T, T), stgA, semsA)

        _wait_gathers(tabs, ix, stgB, semsB)

        @pl.when(k > 0)
        def _():
            pltpu.make_async_copy(
                accB, out.at[pl.ds(tok0 * D, T * D)], semOB).wait()

        _accum_convert(stgB, accB)
        pltpu.async_copy(
            accB, out.at[pl.ds((tok0 + (ca + 1) * T) * D, T * D)], semOB)

        @pl.when(k < NPAIR - 1)
        def _():
            _issue_gathers(tabs, ix, pl.ds((ca + 3) * T, T), stgB, semsB)

        return carry

    lax.fori_loop(0, NPAIR, pair_body, 0)

    pltpu.make_async_copy(accA, out.at[pl.ds(tok0 * D, T * D)], semOA).wait()
    pltpu.make_async_copy(accB, out.at[pl.ds(tok0 * D, T * D)], semOB).wait()


@jax.jit
def _spatial_sc(coords, tlx, brx, tly, bry, ww, wh_t):
    mesh = plsc.VectorSubcoreMesh(
        core_axis_name="c", subcore_axis_name="s", num_cores=NC, num_subcores=NS
    )
    return pl.kernel(
        _body,
        out_type=jax.ShapeDtypeStruct((BL * D,), jnp.float32),
        mesh=mesh,
        scratch_types=[
            pltpu.VMEM((4 * TPW,), jnp.int32),
            pltpu.VMEM((6, TPW), jnp.int32),
            pltpu.VMEM((T * D,), jnp.float32),
            pltpu.VMEM((T * D,), jnp.float32),
            pltpu.VMEM((T, HDP + HDP), jnp.int32),
            pltpu.VMEM((T, HDP + HDP), jnp.int32),
            pltpu.VMEM((T, DP), jnp.int32),
            pltpu.VMEM((T, DP), jnp.int32),
            pltpu.VMEM((T, HDP + HDP), jnp.int32),
            pltpu.VMEM((T, HDP + HDP), jnp.int32),
            pltpu.VMEM((T, DP), jnp.int32),
            pltpu.VMEM((T, DP), jnp.int32),
        ] + [pltpu.SemaphoreType.DMA] * 14,
        compiler_params=pltpu.CompilerParams(needs_layout_passes=False),
    )(coords, tlx, brx, tly, bry, ww, wh_t)


def _bf16_bits(u):
    return (u + 0x7FFF + ((u >> 16) & 1)) >> 16


def _pack_half_block(h):
    r = _bf16_bits(h)
    return lax.bitcast_convert_type(
        r[:, :HDP] | (r[:, HDP:] << 16), jnp.int32)


def _pack_tc_body(tlx, brx, tly, bry, ww, wh_t,
                  otlx, obrx, otly, obry, oww, owh):
    for src, dst, full in ((tlx, otlx, False), (brx, obrx, False),
                           (tly, otly, False), (bry, obry, False),
                           (ww, oww, True), (wh_t, owh, True)):
        u = lax.bitcast_convert_type(src[...], jnp.uint32)
        if full:
            dst[...] = jnp.concatenate(
                [_pack_half_block(u[:, :HD]), _pack_half_block(u[:, HD:])],
                axis=1)
        else:
            dst[...] = _pack_half_block(u)


BR = 256


@jax.jit
def _pack_tc(tlx, brx, tly, bry, ww, wh_t):
    corner_in = pl.BlockSpec((BR, HD), lambda i: (i, 0))
    corner_out = pl.BlockSpec((BR, HDP), lambda i: (i, 0))
    full_in = pl.BlockSpec((BR, D), lambda i: (i, 0))
    full_out = pl.BlockSpec((BR, DP), lambda i: (i, 0))
    return pl.pallas_call(
        _pack_tc_body,
        grid=(V // BR,),
        in_specs=[corner_in] * 4 + [full_in] * 2,
        out_specs=[corner_out] * 4 + [full_out] * 2,
        out_shape=[jax.ShapeDtypeStruct((V, HDP), jnp.int32)] * 4
        + [jax.ShapeDtypeStruct((V, DP), jnp.int32)] * 2,
    )(tlx, brx, tly, bry, ww, wh_t)


def kernel(coordinates, W_tlx, W_brx, W_tly, W_bry, W_w, W_h):
    ptlx, pbrx, ptly, pbry, pww, pwh = _pack_tc(
        W_tlx, W_brx, W_tly, W_bry, W_w, W_h)
    out = _spatial_sc(coordinates.reshape(4 * BL),
                      ptlx, pbrx, ptly, pbry, pww, pwh)
    return out.reshape(B, L, D)


# speedup vs baseline: 2.6662x; 1.0210x over previous
import jax
import jax.numpy as jnp
from jax import lax
from jax.experimental import pallas as pl
from jax.experimental.pallas import tpu as pltpu
from jax.experimental.pallas import tpu_sc as plsc

B, L = 4, 8192
BL = B * L
V = 1024
D = 1024
HD = D // 2

NC, NS = 2, 16
NW = NC * NS
TPW = BL // NW
T = 16
NCHUNK = TPW // T
NPAIR = NCHUNK // 2


def _issue_corners(tabs, ix, sl, c1, s0, sems):
    tlx, brx, tly, bry, ww, wh_t = tabs
    pltpu.async_copy(tlx.at[ix.at[0, sl]], c1.at[:, pl.ds(0, HD)], sems[0])
    pltpu.async_copy(brx.at[ix.at[2, sl]], c1.at[:, pl.ds(HD, HD)], sems[1])
    pltpu.async_copy(tly.at[ix.at[1, sl]], s0.at[:, pl.ds(0, HD)], sems[2])
    pltpu.async_copy(bry.at[ix.at[3, sl]], s0.at[:, pl.ds(HD, HD)], sems[3])


def _wait_corners(tabs, ix, c1, s0, sems):
    tlx, brx, tly, bry, ww, wh_t = tabs
    sl = pl.ds(0, T)
    pltpu.make_async_copy(
        tlx.at[ix.at[0, sl]], c1.at[:, pl.ds(0, HD)], sems[0]).wait()
    pltpu.make_async_copy(
        brx.at[ix.at[2, sl]], c1.at[:, pl.ds(HD, HD)], sems[1]).wait()
    pltpu.make_async_copy(
        tly.at[ix.at[1, sl]], s0.at[:, pl.ds(0, HD)], sems[2]).wait()
    pltpu.make_async_copy(
        bry.at[ix.at[3, sl]], s0.at[:, pl.ds(HD, HD)], sems[3]).wait()


def _sum2(dst, a, b):
    def row_body(r, carry):
        for j in range(D // 16):
            slj = pl.ds(16 * j, 16)
            dst[r, slj] = a[r, slj] + b[r, slj]
        return carry
    lax.fori_loop(0, T, row_body, 0)


def _addto(dst, a):
    def row_body(r, carry):
        for j in range(D // 16):
            slj = pl.ds(16 * j, 16)
            plsc.addupdate(dst.at[r, slj], a[r, slj])
        return carry
    lax.fori_loop(0, T, row_body, 0)


def _body(coords, tlx, brx, tly, bry, ww, wh_t, out,
          cb, ix, accA, accB, c1A, c1B, s0A, s0B, s1, *sems):
    tabs = (tlx, brx, tly, bry, ww, wh_t)
    semsA = sems[0:4]
    semsB = sems[4:8]
    semW = sems[8]
    semHA = sems[9]
    semHB = sems[10]
    semOA = sems[11]
    semOB = sems[12]

    wid = lax.axis_index("s") * NC + lax.axis_index("c")
    tok0 = wid * TPW
    lane = lax.iota(jnp.int32, 16)

    pltpu.sync_copy(coords.at[pl.ds(4 * tok0, 4 * TPW)], cb)

    def dec_body(g, carry):
        pos = 4 * (16 * g + lane)
        x0 = plsc.load_gather(cb, [pos])
        y0 = plsc.load_gather(cb, [pos + 1])
        x1 = plsc.load_gather(cb, [pos + 2])
        y1 = plsc.load_gather(cb, [pos + 3])
        sl = pl.ds(16 * g, 16)
        ix[0, sl] = x0
        ix[1, sl] = y0
        ix[2, sl] = x1
        ix[3, sl] = y1
        ix[4, sl] = x1 - x0
        ix[5, sl] = y1 - y0
        return carry
    lax.fori_loop(0, TPW // 16, dec_body, 0)

    def chunk_sl(c):
        return pl.ds(c * T, T)

    _issue_corners(tabs, ix, chunk_sl(0), c1A, s0A, semsA)
    _issue_corners(tabs, ix, chunk_sl(1), c1B, s0B, semsB)
    pltpu.async_copy(ww.at[ix.at[4, chunk_sl(0)]], s1, semW)

    def phase(k, c, acc, c1, s0, semsC, semH, semO, prefetch):
        @pl.when(k > 0)
        def _():
            pltpu.make_async_copy(
                acc, out.at[pl.ds(tok0, T)], semO).wait()

        _wait_corners(tabs, ix, c1, s0, semsC)
        _sum2(acc, c1, s0)
        pltpu.async_copy(wh_t.at[ix.at[5, chunk_sl(c)]], s0, semH)
        pltpu.make_async_copy(ww.at[ix.at[4, pl.ds(0, T)]], s1, semW).wait()
        _addto(acc, s1)
        @pl.when(c + 1 < NCHUNK)
        def _():
            pltpu.async_copy(ww.at[ix.at[4, chunk_sl(c + 1)]], s1, semW)
        pltpu.make_async_copy(
            wh_t.at[ix.at[5, pl.ds(0, T)]], s0, semH).wait()
        _addto(acc, s0)
        pltpu.async_copy(acc, out.at[pl.ds(tok0 + c * T, T)], semO)
        @pl.when(prefetch)
        def _():
            _issue_corners(tabs, ix, chunk_sl(c + 2), c1, s0, semsC)

    def pair_body(k, carry):
        ca = 2 * k
        phase(k, ca, accA, c1A, s0A, semsA, semHA, semOA, k < NPAIR - 1)
        phase(k, ca + 1, accB, c1B, s0B, semsB, semHB, semOB, k < NPAIR - 1)
        return carry

    lax.fori_loop(0, NPAIR, pair_body, 0)

    pltpu.make_async_copy(accA, out.at[pl.ds(tok0, T)], semOA).wait()
    pltpu.make_async_copy(accB, out.at[pl.ds(tok0, T)], semOB).wait()


@jax.jit
def _spatial_sc(coords, tlx, brx, tly, bry, ww, wh_t):
    mesh = plsc.VectorSubcoreMesh(
        core_axis_name="c", subcore_axis_name="s", num_cores=NC, num_subcores=NS
    )
    return pl.kernel(
        _body,
        out_type=jax.ShapeDtypeStruct((BL, D), jnp.float32),
        mesh=mesh,
        scratch_types=[
            pltpu.VMEM((4 * TPW,), jnp.int32),
            pltpu.VMEM((6, TPW), jnp.int32),
            pltpu.VMEM((T, D), jnp.float32),
            pltpu.VMEM((T, D), jnp.float32),
            pltpu.VMEM((T, D), jnp.float32),
            pltpu.VMEM((T, D), jnp.float32),
            pltpu.VMEM((T, D), jnp.float32),
            pltpu.VMEM((T, D), jnp.float32),
            pltpu.VMEM((T, D), jnp.float32),
        ] + [pltpu.SemaphoreType.DMA] * 13,
        compiler_params=pltpu.CompilerParams(needs_layout_passes=False),
    )(coords, tlx, brx, tly, bry, ww, wh_t)


def kernel(coordinates, W_tlx, W_brx, W_tly, W_bry, W_w, W_h):
    out = _spatial_sc(coordinates.reshape(4 * BL), W_tlx, W_brx, W_tly,
                      W_bry, W_w, W_h)
    return out.reshape(B, L, D)
